# Initial kernel scaffold; baseline (speedup 1.0000x reference)
import jax
import jax.numpy as jnp
from jax.experimental import pallas as pl


def kernel(msg, Wg, bg, W1, b1, W2, b2):
    raise NotImplementedError("write your pallas kernel here")


# speedup vs baseline: 2.6105x; 2.6105x over previous
import functools
import math

import jax
import jax.numpy as jnp
from jax.experimental import pallas as pl
from jax.experimental.pallas import tpu as pltpu

DIM = 1024
HID = 2048
NE = 8
BT = 512
NT = 8192 // BT


def _moe_block(wg_ref, bg_ref, msg_ref, w1_ref, b1_ref, w2_ref, b2_ref,
               out_ref, w_scr):
    e = pl.program_id(1)

    @pl.when(e == 0)
    def _gate():
        xh = msg_ref[...].astype(jnp.bfloat16)
        wh = wg_ref[...].astype(jnp.bfloat16)
        dn = (((1,), (0,)), ((), ()))
        logits = jax.lax.dot_general(
            xh, wh, dn, preferred_element_type=jnp.float32) + bg_ref[...][None, :]
        lane = jax.lax.broadcasted_iota(jnp.int32, (BT, NE), 1)
        cnt = jnp.zeros((BT, NE), jnp.int32)
        for j in range(NE):
            vj = logits[:, j:j + 1]
            beats = (vj > logits) | ((vj == logits) & (j < lane))
            cnt = cnt + beats.astype(jnp.int32)
        sel = cnt < 2
        neg = jnp.full_like(logits, -jnp.inf)
        m = jnp.max(jnp.where(sel, logits, neg), axis=1, keepdims=True)
        ex = jnp.where(sel, jnp.exp(logits - m), 0.0)
        w_scr[...] = ex / jnp.sum(ex, axis=1, keepdims=True)

    x = msg_ref[...].astype(jnp.bfloat16)
    h = jax.lax.dot_general(
        x, w1_ref[0], (((1,), (0,)), ((), ())),
        preferred_element_type=jnp.float32) + b1_ref[0]
    h = 0.5 * h * (1.0 + jax.lax.erf(h * (1.0 / math.sqrt(2.0))))
    y = jax.lax.dot_general(
        h.astype(jnp.bfloat16), w2_ref[0], (((1,), (0,)), ((), ())),
        preferred_element_type=jnp.float32) + b2_ref[0]
    lane_e = jax.lax.broadcasted_iota(jnp.int32, (BT, NE), 1)
    w_e = jnp.sum(jnp.where(lane_e == e, w_scr[...], 0.0), axis=1,
                  keepdims=True)
    wy = w_e * y

    @pl.when(e == 0)
    def _init():
        out_ref[...] = wy

    @pl.when(e != 0)
    def _acc():
        out_ref[...] = out_ref[...] + wy


@jax.jit
def kernel(msg, Wg, bg, W1, b1, W2, b2):
    W1b = W1.astype(jnp.bfloat16)
    W2b = W2.astype(jnp.bfloat16)
    b1r = b1.reshape(NE, 1, HID)
    b2r = b2.reshape(NE, 1, DIM)
    grid = (NT, NE)
    out = pl.pallas_call(
        _moe_block,
        grid=grid,
        in_specs=[
            pl.BlockSpec((DIM, NE), lambda t, e: (0, 0)),
            pl.BlockSpec((NE,), lambda t, e: (0,)),
            pl.BlockSpec((BT, DIM), lambda t, e: (t, 0)),
            pl.BlockSpec((1, DIM, HID), lambda t, e: (e, 0, 0)),
            pl.BlockSpec((1, 1, HID), lambda t, e: (e, 0, 0)),
            pl.BlockSpec((1, HID, DIM), lambda t, e: (e, 0, 0)),
            pl.BlockSpec((1, 1, DIM), lambda t, e: (e, 0, 0)),
        ],
        out_specs=pl.BlockSpec((BT, DIM), lambda t, e: (t, 0)),
        out_shape=jax.ShapeDtypeStruct((8192, DIM), jnp.float32),
        scratch_shapes=[pltpu.VMEM((BT, NE), jnp.float32)],
        compiler_params=pltpu.CompilerParams(
            dimension_semantics=("arbitrary", "arbitrary")),
    )(Wg, bg, msg, W1b, b1r, W2b, b2r)
    return out


# speedup vs baseline: 4.0677x; 1.5582x over previous
import functools
import math

import jax
import jax.numpy as jnp
from jax import lax
from jax.experimental import pallas as pl
from jax.experimental.pallas import tpu as pltpu
from jax.experimental.pallas import tpu_sc as plsc

DIM = 1024
HID = 2048
NE = 8
NT_TOK = 8192
BT = 512
BLK = 256
NBLK = 72
P = NBLK * BLK
NC, NS = 2, 16
NW = NC * NS
TPT = NT_TOK // NW
CH = 64


def _gate_block(msg_ref, wg_ref, bg_ref, e0_ref, e1_ref, w0_ref, w1_ref):
    xh = msg_ref[...].astype(jnp.bfloat16)
    wh = wg_ref[...].astype(jnp.bfloat16)
    logits = jax.lax.dot_general(
        xh, wh, (((1,), (0,)), ((), ())),
        preferred_element_type=jnp.float32) + bg_ref[...][None, :]
    lane = jax.lax.broadcasted_iota(jnp.int32, (BT, NE), 1)
    cnt = jnp.zeros((BT, NE), jnp.int32)
    for j in range(NE):
        vj = logits[:, j:j + 1]
        beats = (vj > logits) | ((vj == logits) & (j < lane))
        cnt = cnt + beats.astype(jnp.int32)
    sel0 = cnt == 0
    sel1 = cnt == 1
    e0_ref[...] = jnp.sum(jnp.where(sel0, lane, 0), axis=1, keepdims=True)
    e1_ref[...] = jnp.sum(jnp.where(sel1, lane, 0), axis=1, keepdims=True)
    l0 = jnp.sum(jnp.where(sel0, logits, 0.0), axis=1, keepdims=True)
    l1 = jnp.sum(jnp.where(sel1, logits, 0.0), axis=1, keepdims=True)
    ex1 = jnp.exp(l1 - l0)
    s = 1.0 + ex1
    w0_ref[...] = 1.0 / s
    w1_ref[...] = ex1 / s


def _gate(msg, Wg, bg):
    nt = NT_TOK // BT
    return pl.pallas_call(
        _gate_block,
        grid=(nt,),
        in_specs=[
            pl.BlockSpec((BT, DIM), lambda t: (t, 0)),
            pl.BlockSpec((DIM, NE), lambda t: (0, 0)),
            pl.BlockSpec((NE,), lambda t: (0,)),
        ],
        out_specs=[
            pl.BlockSpec((BT, 1), lambda t: (t, 0)),
            pl.BlockSpec((BT, 1), lambda t: (t, 0)),
            pl.BlockSpec((BT, 1), lambda t: (t, 0)),
            pl.BlockSpec((BT, 1), lambda t: (t, 0)),
        ],
        out_shape=[
            jax.ShapeDtypeStruct((NT_TOK, 1), jnp.int32),
            jax.ShapeDtypeStruct((NT_TOK, 1), jnp.int32),
            jax.ShapeDtypeStruct((NT_TOK, 1), jnp.float32),
            jax.ShapeDtypeStruct((NT_TOK, 1), jnp.float32),
        ],
    )(msg, Wg, bg)


def _route_block(e0_ref, e1_ref, d0_ref, d1_ref, be_ref):
    e0 = e0_ref[...]
    e1 = e1_ref[...]
    rr = jax.lax.broadcasted_iota(jnp.int32, (128, 128), 0)
    cc = jax.lax.broadcasted_iota(jnp.int32, (128, 128), 1)
    U = (rr <= cc).astype(jnp.bfloat16)
    r64 = jax.lax.broadcasted_iota(jnp.int32, (64, 64), 0)
    c64 = jax.lax.broadcasted_iota(jnp.int32, (64, 64), 1)
    L64 = (c64 < r64).astype(jnp.bfloat16)
    dn = (((1,), (0,)), ((), ()))

    dest0 = jnp.zeros((64, 128), jnp.float32)
    dest1 = jnp.zeros((64, 128), jnp.float32)
    off = 0.0
    blkoffs = []
    for e in range(NE):
        m0 = (e0 == e).astype(jnp.float32)
        m1 = (e1 == e).astype(jnp.float32)
        s = m0 + m1
        incl = jax.lax.dot_general(s.astype(jnp.bfloat16), U, dn,
                                   preferred_element_type=jnp.float32)
        excl = incl - s
        rowtot = incl[:, 127:128]
        roff = jax.lax.dot_general(L64, rowtot.astype(jnp.bfloat16), dn,
                                   preferred_element_type=jnp.float32)
        Epre = excl + roff
        cnt = jnp.sum(s)
        dest0 = dest0 + m0 * (off + Epre)
        dest1 = dest1 + m1 * (off + Epre + m0)
        off = off + jnp.ceil(cnt * (1.0 / BLK)) * BLK
        blkoffs.append(off * (1.0 / BLK))

    lane128 = jax.lax.broadcasted_iota(jnp.int32, (1, 128), 1)
    be = jnp.zeros((1, 128), jnp.int32)
    for e in range(NE):
        be = be + (lane128.astype(jnp.float32) >= blkoffs[e]).astype(jnp.int32)
    be_ref[...] = jnp.minimum(be, NE - 1)
    d0_ref[...] = dest0.astype(jnp.int32)
    d1_ref[...] = dest1.astype(jnp.int32)


def _route(e0_2d, e1_2d):
    return pl.pallas_call(
        _route_block,
        grid=(1,),
        in_specs=[
            pl.BlockSpec((64, 128), lambda i: (0, 0)),
            pl.BlockSpec((64, 128), lambda i: (0, 0)),
        ],
        out_specs=[
            pl.BlockSpec((64, 128), lambda i: (0, 0)),
            pl.BlockSpec((64, 128), lambda i: (0, 0)),
            pl.BlockSpec((1, 128), lambda i: (0, 0)),
        ],
        out_shape=[
            jax.ShapeDtypeStruct((64, 128), jnp.int32),
            jax.ShapeDtypeStruct((64, 128), jnp.int32),
            jax.ShapeDtypeStruct((1, 128), jnp.int32),
        ],
    )(e0_2d, e1_2d)


@functools.lru_cache(maxsize=None)
def _make_dispatch():
    @functools.partial(
        pl.kernel,
        out_type=jax.ShapeDtypeStruct((P, DIM), jnp.float32),
        mesh=plsc.VectorSubcoreMesh(core_axis_name="c", subcore_axis_name="s"),
        scratch_types=[
            pltpu.VMEM((CH, DIM), jnp.float32),
            pltpu.VMEM((CH,), jnp.int32),
            pltpu.VMEM((CH,), jnp.int32),
            pltpu.SemaphoreType.DMA,
            pltpu.SemaphoreType.DMA,
        ],
    )
    def dispatch(msg_hbm, d0_hbm, d1_hbm, xs_hbm, rows_v, i0_v, i1_v, s0, s1):
        wid = lax.axis_index("c") * NS + lax.axis_index("s")
        base0 = pl.multiple_of(wid * TPT, TPT)
        for ci in range(TPT // CH):
            base = pl.multiple_of(base0 + ci * CH, CH)
            pltpu.sync_copy(msg_hbm.at[pl.ds(base, CH)], rows_v)
            pltpu.sync_copy(d0_hbm.at[pl.ds(base, CH)], i0_v)
            pltpu.sync_copy(d1_hbm.at[pl.ds(base, CH)], i1_v)
            c0 = pltpu.async_copy(rows_v, xs_hbm.at[i0_v], s0)
            c1 = pltpu.async_copy(rows_v, xs_hbm.at[i1_v], s1)
            c0.wait()
            c1.wait()

    return dispatch


def _dispatch(msg, d0, d1):
    return _make_dispatch()(msg, d0, d1)


def _ffn_block(be_ref, xs_ref, w1_ref, b1_ref, w2_ref, b2_ref, ys_ref):
    x = xs_ref[...].astype(jnp.bfloat16)
    h = jax.lax.dot_general(
        x, w1_ref[0], (((1,), (0,)), ((), ())),
        preferred_element_type=jnp.float32) + b1_ref[0]
    h = 0.5 * h * (1.0 + jax.lax.erf(h * (1.0 / math.sqrt(2.0))))
    ys_ref[...] = jax.lax.dot_general(
        h.astype(jnp.bfloat16), w2_ref[0], (((1,), (0,)), ((), ())),
        preferred_element_type=jnp.float32) + b2_ref[0]


def _ffn(block_expert, xs, W1b, b1r, W2b, b2r):
    grid_spec = pltpu.PrefetchScalarGridSpec(
        num_scalar_prefetch=1,
        grid=(NBLK,),
        in_specs=[
            pl.BlockSpec((BLK, DIM), lambda b, be: (b, 0)),
            pl.BlockSpec((1, DIM, HID), lambda b, be: (be[b], 0, 0)),
            pl.BlockSpec((1, 1, HID), lambda b, be: (be[b], 0, 0)),
            pl.BlockSpec((1, HID, DIM), lambda b, be: (be[b], 0, 0)),
            pl.BlockSpec((1, 1, DIM), lambda b, be: (be[b], 0, 0)),
        ],
        out_specs=pl.BlockSpec((BLK, DIM), lambda b, be: (b, 0)),
    )
    return pl.pallas_call(
        _ffn_block,
        grid_spec=grid_spec,
        out_shape=jax.ShapeDtypeStruct((P, DIM), jnp.float32),
    )(block_expert, xs, W1b, b1r, W2b, b2r)


@functools.lru_cache(maxsize=None)
def _make_permute():
    @functools.partial(
        pl.kernel,
        out_type=[
            jax.ShapeDtypeStruct((NT_TOK, DIM), jnp.float32),
            jax.ShapeDtypeStruct((NT_TOK, DIM), jnp.float32),
        ],
        mesh=plsc.VectorSubcoreMesh(core_axis_name="c", subcore_axis_name="s"),
        scratch_types=[
            pltpu.VMEM((CH, DIM), jnp.float32),
            pltpu.VMEM((CH,), jnp.int32),
            pltpu.SemaphoreType.DMA,
        ],
    )
    def permute(ys_hbm, d0_hbm, d1_hbm, y0_hbm, y1_hbm, rows_v, idx_v, sem):
        wid = lax.axis_index("c") * NS + lax.axis_index("s")
        base0 = pl.multiple_of(wid * TPT, TPT)
        for ci in range(TPT // CH):
            base = pl.multiple_of(base0 + ci * CH, CH)
            pltpu.sync_copy(d0_hbm.at[pl.ds(base, CH)], idx_v)
            pltpu.async_copy(ys_hbm.at[idx_v], rows_v, sem).wait()
            pltpu.sync_copy(rows_v, y0_hbm.at[pl.ds(base, CH)])
            pltpu.sync_copy(d1_hbm.at[pl.ds(base, CH)], idx_v)
            pltpu.async_copy(ys_hbm.at[idx_v], rows_v, sem).wait()
            pltpu.sync_copy(rows_v, y1_hbm.at[pl.ds(base, CH)])

    return permute


def _permute(ys, d0, d1):
    return _make_permute()(ys, d0, d1)


def _combine_block(y0_ref, y1_ref, w0_ref, w1_ref, out_ref):
    out_ref[...] = w0_ref[...] * y0_ref[...] + w1_ref[...] * y1_ref[...]


def _combine(y0, y1, w0, w1):
    nt = NT_TOK // BT
    return pl.pallas_call(
        _combine_block,
        grid=(nt,),
        in_specs=[
            pl.BlockSpec((BT, DIM), lambda t: (t, 0)),
            pl.BlockSpec((BT, DIM), lambda t: (t, 0)),
            pl.BlockSpec((BT, 1), lambda t: (t, 0)),
            pl.BlockSpec((BT, 1), lambda t: (t, 0)),
        ],
        out_specs=pl.BlockSpec((BT, DIM), lambda t: (t, 0)),
        out_shape=jax.ShapeDtypeStruct((NT_TOK, DIM), jnp.float32),
    )(y0, y1, w0, w1)


@jax.jit
def kernel(msg, Wg, bg, W1, b1, W2, b2):
    W1b = W1.astype(jnp.bfloat16)
    W2b = W2.astype(jnp.bfloat16)
    b1r = b1.reshape(NE, 1, HID)
    b2r = b2.reshape(NE, 1, DIM)

    e0, e1, w0, w1 = _gate(msg, Wg, bg)
    d0_2d, d1_2d, be_2d = _route(e0.reshape(64, 128), e1.reshape(64, 128))
    d0 = d0_2d.reshape(NT_TOK)
    d1 = d1_2d.reshape(NT_TOK)
    block_expert = be_2d.reshape(128)[:NBLK]

    xs = _dispatch(msg, d0, d1)
    ys = _ffn(block_expert, xs, W1b, b1r, W2b, b2r)
    y0, y1 = _permute(ys, d0, d1)
    return _combine(y0, y1, w0, w1)
